# baseline (speedup 1.0000x reference)
import functools

import jax
import jax.numpy as jnp
from jax import lax
from jax.experimental import pallas as pl
from jax.experimental.pallas import tpu as pltpu
from jax.experimental.pallas import tpu_sc as plsc

B = 16384
D = 128
NUM_REL = 1000

_info = plsc.get_sparse_core_info()
NC, NS, L = _info.num_cores, _info.num_subcores, _info.num_lanes
NW = NC * NS
B_PER_W = B // NW
CHUNK = 64
N_CHUNKS = B_PER_W // CHUNK
NBUF = 4
GROUPS = CHUNK // L
DSL = D // L


def _sc_kernel():
    mesh = plsc.VectorSubcoreMesh(core_axis_name="c", subcore_axis_name="s")

    @functools.partial(
        pl.kernel,
        mesh=mesh,
        out_type=jax.ShapeDtypeStruct((B,), jnp.float32),
        scratch_types=[
            pltpu.VMEM((B_PER_W,), jnp.int32),
            pltpu.VMEM((NBUF, CHUNK, D), jnp.float32),
            pltpu.VMEM((NBUF, CHUNK, D), jnp.float32),
            pltpu.VMEM((NBUF, CHUNK, D), jnp.float32),
            pltpu.VMEM((N_CHUNKS, CHUNK), jnp.float32),
            pltpu.VMEM_SHARED((NUM_REL, D), jnp.float32),
        ] + [pltpu.SemaphoreType.DMA] * (3 * NBUF + 1),
    )
    def k(src_hbm, ids_hbm, dst_hbm, table_hbm, out_hbm,
          idx_v, rel_v, src_v, dst_v, out_v, table_sh, *all_sems):
        wid = lax.axis_index("s") * NC + lax.axis_index("c")
        sid = lax.axis_index("s")
        base = wid * B_PER_W
        lanes = lax.iota(jnp.int32, L)
        sems = [all_sems[3 * i:3 * i + 3] for i in range(NBUF)]
        osem = all_sems[3 * NBUF]

        dnums = lax.GatherDimensionNumbers(
            offset_dims=(), collapsed_slice_dims=(0,), start_index_map=(0,))

        def lane_perm(x, perm):
            return lax.gather(
                x, perm[:, None], dimension_numbers=dnums, slice_sizes=(1,),
                mode=lax.GatherScatterMode.PROMISE_IN_BOUNDS)

        def lane_sum(x):
            for m in (8, 4, 2, 1):
                x = x + lane_perm(x, jnp.bitwise_xor(lanes, m))
            return x

        def fire_linear(c):
            bb = c % NBUF
            _, s, d = sems[bb]
            rb = base + c * CHUNK
            return (
                pltpu.async_copy(src_hbm.at[pl.ds(rb, CHUNK)],
                                 src_v.at[bb], s),
                pltpu.async_copy(dst_hbm.at[pl.ds(rb, CHUNK)],
                                 dst_v.at[bb], d),
            )

        def fire_gather(c, table):
            bb = c % NBUF
            return pltpu.async_copy(
                table.at[idx_v.at[pl.ds(c * CHUNK, CHUNK)]],
                rel_v.at[bb], sems[bb][0])

        pltpu.sync_copy(ids_hbm.at[pl.ds(base, B_PER_W)], idx_v)
        inflight = {0: fire_linear(0) + (fire_gather(0, table_hbm),),
                    1: fire_linear(1), 2: fire_linear(2)}

        @pl.when(sid < 15)
        def _():
            rslab = sid * 64
            pltpu.sync_copy(table_hbm.at[pl.ds(rslab, 64)],
                            table_sh.at[pl.ds(rslab, 64)])

        @pl.when(sid == 15)
        def _():
            pltpu.sync_copy(table_hbm.at[pl.ds(960, 40)],
                            table_sh.at[pl.ds(960, 40)])

        plsc.subcore_barrier()
        inflight[1] = inflight[1] + (fire_gather(1, table_sh),)
        inflight[2] = inflight[2] + (fire_gather(2, table_sh),)

        outs = []
        for c in range(N_CHUNKS):
            bb = c % NBUF
            if c + 3 < N_CHUNKS:
                inflight[c + 3] = (fire_linear(c + 3)
                                   + (fire_gather(c + 3, table_sh),))
            for h in inflight.pop(c):
                h.wait()

            def group_body(g, _):
                r0 = g * L

                def row_body(i, res):
                    r = r0 + i
                    acc0 = acc1 = None
                    for j in range(DSL):
                        sl = pl.ds(j * L, L)
                        p = (src_v[bb, r, sl]
                             * rel_v[bb, r, sl]
                             * dst_v[bb, r, sl])
                        if j % 2 == 0:
                            acc0 = p if acc0 is None else acc0 + p
                        else:
                            acc1 = p if acc1 is None else acc1 + p
                    tot = lane_sum(acc0 + acc1)
                    return jnp.where(lanes == i, tot, res)

                res = lax.fori_loop(0, L, row_body,
                                    jnp.zeros((L,), jnp.float32))
                out_v[c, pl.ds(r0, L)] = res
                return 0

            lax.fori_loop(0, GROUPS, group_body, 0)
            outs.append(pltpu.async_copy(
                out_v.at[c], out_hbm.at[pl.ds(base + c * CHUNK, CHUNK)],
                osem))

        for h in outs:
            h.wait()

    return k


_scorer = _sc_kernel()


@jax.jit
def kernel(src_emb, rel_ids, dst_emb, rel_emb_table):
    ids = rel_ids.astype(jnp.int32)
    return _scorer(src_emb, ids, dst_emb, rel_emb_table)


# speedup vs baseline: 1.0325x; 1.0325x over previous
import functools

import jax
import jax.numpy as jnp
from jax import lax
from jax.experimental import pallas as pl
from jax.experimental.pallas import tpu as pltpu
from jax.experimental.pallas import tpu_sc as plsc

B = 16384
D = 128
NUM_REL = 1000

_info = plsc.get_sparse_core_info()
NC, NS, L = _info.num_cores, _info.num_subcores, _info.num_lanes
NW = NC * NS
B_PER_W = B // NW
CHUNK = 128
SIZES = (32, 96, 128, 128, 128)
OFFS = (0, 32, 128, 256, 384)
DSL = D // L

assert sum(SIZES) == B_PER_W


def _sc_kernel():
    mesh = plsc.VectorSubcoreMesh(core_axis_name="c", subcore_axis_name="s")

    @functools.partial(
        pl.kernel,
        mesh=mesh,
        out_type=jax.ShapeDtypeStruct((B,), jnp.float32),
        scratch_types=[
            pltpu.VMEM((B_PER_W,), jnp.int32),
            pltpu.VMEM((2, CHUNK, D), jnp.float32),
            pltpu.VMEM((2, CHUNK, D), jnp.float32),
            pltpu.VMEM((2, CHUNK, D), jnp.float32),
            pltpu.VMEM((B_PER_W,), jnp.float32),
            pltpu.VMEM_SHARED((NUM_REL, D), jnp.float32),
            pltpu.SemaphoreType.DMA,
            pltpu.SemaphoreType.DMA,
            pltpu.SemaphoreType.DMA,
            pltpu.SemaphoreType.DMA,
            pltpu.SemaphoreType.DMA,
            pltpu.SemaphoreType.DMA,
            pltpu.SemaphoreType.DMA,
        ],
    )
    def k(src_hbm, ids_hbm, dst_hbm, table_hbm, out_hbm,
          idx_v, rel_v, src_v, dst_v, out_v, table_sh,
          gs0, ss0, ds0, gs1, ss1, ds1, osem):
        wid = lax.axis_index("s") * NC + lax.axis_index("c")
        sid = lax.axis_index("s")
        base = wid * B_PER_W
        lanes = lax.iota(jnp.int32, L)
        sems = [(gs0, ss0, ds0), (gs1, ss1, ds1)]

        dnums = lax.GatherDimensionNumbers(
            offset_dims=(), collapsed_slice_dims=(0,), start_index_map=(0,))

        def lane_perm(x, perm):
            return lax.gather(
                x, perm[:, None], dimension_numbers=dnums, slice_sizes=(1,),
                mode=lax.GatherScatterMode.PROMISE_IN_BOUNDS)

        def lane_sum(x):
            for m in (8, 4, 2, 1):
                x = x + lane_perm(x, jnp.bitwise_xor(lanes, m))
            return x

        def fire_linear(c):
            bb = c % 2
            _, s, d = sems[bb]
            rb = base + OFFS[c]
            n = SIZES[c]
            return (
                pltpu.async_copy(src_hbm.at[pl.ds(rb, n)],
                                 src_v.at[bb].at[pl.ds(0, n)], s),
                pltpu.async_copy(dst_hbm.at[pl.ds(rb, n)],
                                 dst_v.at[bb].at[pl.ds(0, n)], d),
            )

        def fire_gather(c, table):
            bb = c % 2
            return pltpu.async_copy(
                table.at[idx_v.at[pl.ds(OFFS[c], SIZES[c])]],
                rel_v.at[bb].at[pl.ds(0, SIZES[c])], sems[bb][0])

        pltpu.sync_copy(ids_hbm.at[pl.ds(base, B_PER_W)], idx_v)
        inflight = fire_linear(0) + (fire_gather(0, table_hbm),)

        @pl.when(sid < 15)
        def _():
            rslab = sid * 64
            pltpu.sync_copy(table_hbm.at[pl.ds(rslab, 64)],
                            table_sh.at[pl.ds(rslab, 64)])

        @pl.when(sid == 15)
        def _():
            pltpu.sync_copy(table_hbm.at[pl.ds(960, 40)],
                            table_sh.at[pl.ds(960, 40)])

        plsc.subcore_barrier()

        outs = []
        for c in range(len(SIZES)):
            bb = c % 2
            nxt = None
            if c + 1 < len(SIZES):
                nxt = fire_linear(c + 1) + (fire_gather(c + 1, table_sh),)
            for h in inflight:
                h.wait()
            inflight = nxt

            def group_body(g, _):
                r0 = g * L

                def row_body(i, res):
                    r = r0 + i
                    acc0 = acc1 = None
                    for j in range(DSL):
                        sl = pl.ds(j * L, L)
                        p = (src_v[bb, r, sl]
                             * rel_v[bb, r, sl]
                             * dst_v[bb, r, sl])
                        if j % 2 == 0:
                            acc0 = p if acc0 is None else acc0 + p
                        else:
                            acc1 = p if acc1 is None else acc1 + p
                    tot = lane_sum(acc0 + acc1)
                    return jnp.where(lanes == i, tot, res)

                res = lax.fori_loop(0, L, row_body,
                                    jnp.zeros((L,), jnp.float32))
                out_v[pl.ds(OFFS[c] + r0, L)] = res
                return 0

            lax.fori_loop(0, SIZES[c] // L, group_body, 0)
            outs.append(pltpu.async_copy(
                out_v.at[pl.ds(OFFS[c], SIZES[c])],
                out_hbm.at[pl.ds(base + OFFS[c], SIZES[c])], osem))

        for h in outs:
            h.wait()

    return k


_scorer = _sc_kernel()


@jax.jit
def kernel(src_emb, rel_ids, dst_emb, rel_emb_table):
    ids = rel_ids.astype(jnp.int32)
    return _scorer(src_emb, ids, dst_emb, rel_emb_table)


# speedup vs baseline: 1.0499x; 1.0168x over previous
import functools

import jax
import jax.numpy as jnp
from jax import lax
from jax.experimental import pallas as pl
from jax.experimental.pallas import tpu as pltpu
from jax.experimental.pallas import tpu_sc as plsc

B = 16384
D = 128
NUM_REL = 1000

_info = plsc.get_sparse_core_info()
NC, NS, L = _info.num_cores, _info.num_subcores, _info.num_lanes
NW = NC * NS
B_PER_W = B // NW
CHUNK = 128
N_CHUNKS = B_PER_W // CHUNK
GROUPS = CHUNK // L
DSL = D // L


def _sc_kernel():
    mesh = plsc.VectorSubcoreMesh(core_axis_name="c", subcore_axis_name="s")

    @functools.partial(
        pl.kernel,
        mesh=mesh,
        out_type=jax.ShapeDtypeStruct((B,), jnp.float32),
        scratch_types=[
            pltpu.VMEM((B_PER_W,), jnp.int32),
            pltpu.VMEM((2, CHUNK, D), jnp.float32),
            pltpu.VMEM((2, CHUNK, D), jnp.float32),
            pltpu.VMEM((2, CHUNK, D), jnp.float32),
            pltpu.VMEM((2, CHUNK), jnp.float32),
            pltpu.VMEM_SHARED((NUM_REL, D), jnp.float32),
            pltpu.SemaphoreType.DMA,
            pltpu.SemaphoreType.DMA,
            pltpu.SemaphoreType.DMA,
            pltpu.SemaphoreType.DMA,
            pltpu.SemaphoreType.DMA,
            pltpu.SemaphoreType.DMA,
        ],
    )
    def k(src_hbm, ids_hbm, dst_hbm, table_hbm, out_hbm,
          idx_v, rel_v, src_v, dst_v, out_v, table_sh,
          gs0, ss0, ds0, gs1, ss1, ds1):
        wid = lax.axis_index("s") * NC + lax.axis_index("c")
        sid = lax.axis_index("s")
        base = wid * B_PER_W
        lanes = lax.iota(jnp.int32, L)
        sems = [(gs0, ss0, ds0), (gs1, ss1, ds1)]

        dnums = lax.GatherDimensionNumbers(
            offset_dims=(), collapsed_slice_dims=(0,), start_index_map=(0,))

        def lane_perm(x, perm):
            return lax.gather(
                x, perm[:, None], dimension_numbers=dnums, slice_sizes=(1,),
                mode=lax.GatherScatterMode.PROMISE_IN_BOUNDS)

        def lane_sum(x):
            for m in (8, 4, 2, 1):
                x = x + lane_perm(x, jnp.bitwise_xor(lanes, m))
            return x

        def fire_linear(c):
            bb = c % 2
            _, s, d = sems[bb]
            rb = base + c * CHUNK
            return (
                pltpu.async_copy(src_hbm.at[pl.ds(rb, CHUNK)],
                                 src_v.at[bb], s),
                pltpu.async_copy(dst_hbm.at[pl.ds(rb, CHUNK)],
                                 dst_v.at[bb], d),
            )

        def fire_gather(c):
            bb = c % 2
            return pltpu.async_copy(
                table_sh.at[idx_v.at[pl.ds(c * CHUNK, CHUNK)]],
                rel_v.at[bb], sems[bb][0])

        lin0 = fire_linear(0)
        pltpu.sync_copy(ids_hbm.at[pl.ds(base, B_PER_W)], idx_v)

        @pl.when(sid < 15)
        def _():
            rslab = sid * 64
            pltpu.sync_copy(table_hbm.at[pl.ds(rslab, 64)],
                            table_sh.at[pl.ds(rslab, 64)])

        @pl.when(sid == 15)
        def _():
            pltpu.sync_copy(table_hbm.at[pl.ds(960, 40)],
                            table_sh.at[pl.ds(960, 40)])

        plsc.subcore_barrier()
        inflight = (fire_gather(0),) + lin0

        for c in range(N_CHUNKS):
            bb = c % 2
            nxt = None
            if c + 1 < N_CHUNKS:
                nxt = (fire_gather(c + 1),) + fire_linear(c + 1)
            for h in inflight:
                h.wait()
            inflight = nxt

            def group_body(g, _):
                r0 = g * L

                def row_body(i, res):
                    r = r0 + i
                    acc0 = acc1 = None
                    for j in range(DSL):
                        sl = pl.ds(j * L, L)
                        p = (src_v[bb, r, sl]
                             * rel_v[bb, r, sl]
                             * dst_v[bb, r, sl])
                        if j % 2 == 0:
                            acc0 = p if acc0 is None else acc0 + p
                        else:
                            acc1 = p if acc1 is None else acc1 + p
                    tot = lane_sum(acc0 + acc1)
                    return jnp.where(lanes == i, tot, res)

                res = lax.fori_loop(0, L, row_body,
                                    jnp.zeros((L,), jnp.float32))
                out_v[bb, pl.ds(r0, L)] = res
                return 0

            lax.fori_loop(0, GROUPS, group_body, 0)
            pltpu.sync_copy(out_v.at[bb],
                            out_hbm.at[pl.ds(base + c * CHUNK, CHUNK)])

    return k


_scorer = _sc_kernel()


@jax.jit
def kernel(src_emb, rel_ids, dst_emb, rel_emb_table):
    ids = rel_ids.astype(jnp.int32)
    return _scorer(src_emb, ids, dst_emb, rel_emb_table)
